# baseline (speedup 1.0000x reference)
import functools

import jax
import jax.numpy as jnp
from jax import lax
from jax.experimental import pallas as pl
from jax.experimental.pallas import tpu as pltpu
from jax.experimental.pallas import tpu_sc as plsc

NN = 10000
EE = 320000
DH = 128
DL = 64

NP = 10112
NC, NS = 2, 16
NW = NC * NS
EPW = EE // NW
CH = 80
NCHUNK = EPW // CH
RPT = NP // NS

_mesh = plsc.VectorSubcoreMesh(
    core_axis_name="c", subcore_axis_name="s", num_cores=NC, num_subcores=NS)


def _zero_fill(buf, rows, width):
    zeros16 = jnp.zeros((16,), jnp.float32)

    def zrow(i, _):
        for q in range(width // 16):
            buf[i, pl.ds(q * 16, 16)] = zeros16
        return 0

    lax.fori_loop(0, rows, zrow, 0)


def _tile_copy(src_buf, dst_view, row_base):
    for k in range(RPT // CH):
        pltpu.sync_copy(src_buf, dst_view.at[pl.ds(row_base + k * CH, CH)])
    rem = RPT % CH
    pltpu.sync_copy(src_buf.at[pl.ds(0, rem)],
                    dst_view.at[pl.ds(row_base + (RPT // CH) * CH, rem)])


DEGW = DH


@functools.partial(
    pl.kernel,
    out_type=jax.ShapeDtypeStruct((NC, NP, DEGW), jnp.float32),
    mesh=_mesh,
    scratch_types=[
        pltpu.VMEM_SHARED((NP, DEGW), jnp.float32),
        pltpu.VMEM((NCHUNK, CH), jnp.int32),
        pltpu.VMEM((CH, DEGW), jnp.float32),
        pltpu.SemaphoreType.DMA,
    ],
)
def _deg_kernel(dst_hbm, out_hbm, acc, dst_l, ones_l, ssem):
    c = lax.axis_index("c")
    s = lax.axis_index("s")
    wid = c * NS + s
    row_base = pl.multiple_of(s * RPT, 8)

    _zero_fill(ones_l, CH, DEGW)
    _tile_copy(ones_l, acc, row_base)

    ones16 = jnp.ones((16,), jnp.float32)

    def orow(i, _):
        for q in range(DEGW // 16):
            ones_l[i, pl.ds(q * 16, 16)] = ones16
        return 0

    lax.fori_loop(0, CH, orow, 0)
    pltpu.sync_copy(dst_hbm.at[wid], dst_l)
    plsc.subcore_barrier()

    def wait_one(_i, _):
        pltpu.make_async_copy(ones_l, acc.at[dst_l.at[0]], ssem).wait()
        return 0

    def body(j, _):
        pltpu.async_copy(ones_l, acc.at[dst_l.at[j]], ssem, add=True)

        @pl.when(j >= 8)
        def _():
            wait_one(0, 0)

        return 0

    lax.fori_loop(0, NCHUNK, body, 0)
    lax.fori_loop(0, 8, wait_one, 0)
    plsc.subcore_barrier()
    pltpu.sync_copy(acc.at[pl.ds(row_base, RPT)],
                    out_hbm.at[c, pl.ds(row_base, RPT)])


@functools.partial(
    pl.kernel,
    out_type=jax.ShapeDtypeStruct((NC, NP, DH), jnp.float32),
    mesh=_mesh,
    scratch_types=[
        pltpu.VMEM_SHARED((NP, DH), jnp.float32),
        pltpu.VMEM((4, CH, DH), jnp.float32),
        pltpu.VMEM((8, 2, CH), jnp.int32),
        pltpu.SemaphoreType.DMA((4,)),
        pltpu.SemaphoreType.DMA((8,)),
    ],
)
def _prop_kernel(g_hbm, src_hbm, dst_hbm, out_hbm,
                 acc, rows, idxb, gsem, isem):
    c = lax.axis_index("c")
    s = lax.axis_index("s")
    wid = c * NS + s
    row_base = pl.multiple_of(s * RPT, 8)

    _zero_fill(rows.at[0], CH, DH)
    _tile_copy(rows.at[0], acc, row_base)
    plsc.subcore_barrier()

    ebase = wid * EPW

    def stage(j, q):
        off = pl.multiple_of(ebase + j * CH, 8)
        pltpu.async_copy(src_hbm.at[pl.ds(off, CH)], idxb.at[q, 0],
                         isem.at[q])
        pltpu.async_copy(dst_hbm.at[pl.ds(off, CH)], idxb.at[q, 1],
                         isem.at[q])

    def wait_stage(q):
        pltpu.make_async_copy(src_hbm.at[pl.ds(0, CH)], idxb.at[q, 0],
                              isem.at[q]).wait()
        pltpu.make_async_copy(src_hbm.at[pl.ds(0, CH)], idxb.at[q, 1],
                              isem.at[q]).wait()

    def gather(b, q):
        pltpu.async_copy(g_hbm.at[idxb.at[q, 0]], rows.at[b], gsem.at[b])

    def wait_gather(b):
        pltpu.make_async_copy(g_hbm.at[idxb.at[0, 0]], rows.at[b],
                              gsem.at[b]).wait()

    def scatter(b, q):
        pltpu.sync_copy(rows.at[b], acc.at[idxb.at[q, 1]], add=True)

    for q in range(8):
        stage(q, q)
    for b in range(4):
        wait_stage(b)
        gather(b, b)

    def body(i, _):
        j8 = i * 8
        for k in range(8):
            j = j8 + k
            b = k % 4
            q = k

            @pl.when(j < NCHUNK)
            def _():
                wait_gather(b)
                scatter(b, q)

            @pl.when(j + 8 < NCHUNK)
            def _():
                stage(j + 8, q)

            @pl.when(j + 4 < NCHUNK)
            def _():
                wait_stage((k + 4) % 8)
                gather(b, (k + 4) % 8)

        return 0

    lax.fori_loop(0, (NCHUNK + 7) // 8, body, 0)
    plsc.subcore_barrier()
    pltpu.sync_copy(acc.at[pl.ds(row_base, RPT)],
                    out_hbm.at[c, pl.ds(row_base, RPT)])


def _mm1_body(x_ref, w1_ref, h1_ref):
    h1_ref[...] = jnp.dot(x_ref[...], w1_ref[...],
                          preferred_element_type=jnp.float32)


def _pre_body(degp_ref, h1_ref, g1_ref, dinv_ref):
    degp = degp_ref[...]
    deg = 1.0 + degp[0, :NN, 0:1] + degp[1, :NN, 0:1]
    dinv = lax.rsqrt(deg)
    g1_ref[...] = h1_ref[...] * dinv
    dinv_ref[...] = dinv


def _mid_body(sp_ref, g1_ref, dinv_ref, b1_ref, gamma_ref, beta_ref, g2_ref):
    sp = sp_ref[...]
    dinv = jnp.broadcast_to(dinv_ref[...], (NN, DH))
    p = dinv * (sp[0, :NN] + sp[1, :NN] + g1_ref[...]) + b1_ref[...]
    m = jnp.mean(p, axis=0, keepdims=True)
    v = jnp.mean(p * p, axis=0, keepdims=True) - m * m
    h = (p - m) / jnp.sqrt(v + 1e-5) * gamma_ref[...] + beta_ref[...]
    h = jnp.maximum(h, 0.0)
    g2_ref[...] = h * dinv


def _post_body(sp_ref, g2_ref, dinv_ref, wmu_ref, bmu_ref, wlv_ref, blv_ref,
               zm_ref, zl_ref):
    sp = sp_ref[...]
    q = jnp.broadcast_to(dinv_ref[...], (NN, DH)) * (
        sp[0, :NN] + sp[1, :NN] + g2_ref[...])
    zm_ref[...] = jnp.dot(q, wmu_ref[...],
                          preferred_element_type=jnp.float32) + bmu_ref[...]
    zl_ref[...] = jnp.dot(q, wlv_ref[...],
                          preferred_element_type=jnp.float32) + blv_ref[...]


_mm1_call = pl.pallas_call(
    _mm1_body,
    out_shape=jax.ShapeDtypeStruct((NN, DH), jnp.float32),
)

_pre_call = pl.pallas_call(
    _pre_body,
    out_shape=[jax.ShapeDtypeStruct((NN, DH), jnp.float32),
               jax.ShapeDtypeStruct((NN, 1), jnp.float32)],
)

_mid_call = pl.pallas_call(
    _mid_body,
    out_shape=jax.ShapeDtypeStruct((NN, DH), jnp.float32),
)

_post_call = pl.pallas_call(
    _post_body,
    out_shape=[jax.ShapeDtypeStruct((NN, DL), jnp.float32),
               jax.ShapeDtypeStruct((NN, DL), jnp.float32)],
)


@jax.jit
def kernel(x, edge_index, y, W1, b1, gamma1, beta1, Wmu, bmu, Wlv, blv):
    src_flat = edge_index[:, 0].astype(jnp.int32)
    dst_flat = edge_index[:, 1].astype(jnp.int32)
    dst = dst_flat.reshape(NW, NCHUNK, CH)
    h1 = _mm1_call(x, W1)
    degp = _deg_kernel(dst)
    g1, dinv = _pre_call(degp, h1)
    s1 = _prop_kernel(g1, src_flat, dst_flat)
    g2 = _mid_call(s1, g1, dinv, b1.reshape(1, DH), gamma1.reshape(1, DH),
                   beta1.reshape(1, DH))
    s2 = _prop_kernel(g2, src_flat, dst_flat)
    zm, zl = _post_call(s2, g2, dinv, Wmu, bmu.reshape(1, DL), Wlv,
                        blv.reshape(1, DL))
    return (zm, zl)


# speedup vs baseline: 1.0026x; 1.0026x over previous
import functools

import jax
import jax.numpy as jnp
from jax import lax
from jax.experimental import pallas as pl
from jax.experimental.pallas import tpu as pltpu
from jax.experimental.pallas import tpu_sc as plsc

NN = 10000
EE = 320000
DH = 128
DL = 64

NP = 10112
NC, NS = 2, 16
NW = NC * NS
EPW = EE // NW
CH = 80
NCHUNK = EPW // CH
RPT = NP // NS

_mesh = plsc.VectorSubcoreMesh(
    core_axis_name="c", subcore_axis_name="s", num_cores=NC, num_subcores=NS)


def _zero_fill(buf, rows, width):
    zeros16 = jnp.zeros((16,), jnp.float32)

    def zrow(i, _):
        for q in range(width // 16):
            buf[i, pl.ds(q * 16, 16)] = zeros16
        return 0

    lax.fori_loop(0, rows, zrow, 0)


def _tile_copy(src_buf, dst_view, row_base):
    for k in range(RPT // CH):
        pltpu.sync_copy(src_buf, dst_view.at[pl.ds(row_base + k * CH, CH)])
    rem = RPT % CH
    pltpu.sync_copy(src_buf.at[pl.ds(0, rem)],
                    dst_view.at[pl.ds(row_base + (RPT // CH) * CH, rem)])


DEGW = DH


@functools.partial(
    pl.kernel,
    out_type=jax.ShapeDtypeStruct((NC, NP, DEGW), jnp.float32),
    mesh=_mesh,
    scratch_types=[
        pltpu.VMEM_SHARED((NP, DEGW), jnp.float32),
        pltpu.VMEM((NCHUNK, CH), jnp.int32),
        pltpu.VMEM((CH, DEGW), jnp.float32),
        pltpu.SemaphoreType.DMA,
    ],
)
def _deg_kernel(dst_hbm, out_hbm, acc, dst_l, ones_l, ssem):
    c = lax.axis_index("c")
    s = lax.axis_index("s")
    wid = c * NS + s
    row_base = pl.multiple_of(s * RPT, 8)

    _zero_fill(ones_l, CH, DEGW)
    _tile_copy(ones_l, acc, row_base)

    ones16 = jnp.ones((16,), jnp.float32)

    def orow(i, _):
        for q in range(DEGW // 16):
            ones_l[i, pl.ds(q * 16, 16)] = ones16
        return 0

    lax.fori_loop(0, CH, orow, 0)
    pltpu.sync_copy(dst_hbm.at[wid], dst_l)
    plsc.subcore_barrier()

    def wait_one(_i, _):
        pltpu.make_async_copy(ones_l, acc.at[dst_l.at[0]], ssem).wait()
        return 0

    def body(j, _):
        pltpu.async_copy(ones_l, acc.at[dst_l.at[j]], ssem, add=True)

        @pl.when(j >= 8)
        def _():
            wait_one(0, 0)

        return 0

    lax.fori_loop(0, NCHUNK, body, 0)
    lax.fori_loop(0, 8, wait_one, 0)
    plsc.subcore_barrier()
    pltpu.sync_copy(acc.at[pl.ds(row_base, RPT)],
                    out_hbm.at[c, pl.ds(row_base, RPT)])


@functools.partial(
    pl.kernel,
    out_type=jax.ShapeDtypeStruct((NC, NP, DH), jnp.float32),
    mesh=_mesh,
    scratch_types=[
        pltpu.VMEM_SHARED((NP, DH), jnp.float32),
        pltpu.VMEM((4, CH, DH), jnp.float32),
        pltpu.VMEM((8, 2, CH), jnp.int32),
        pltpu.SemaphoreType.DMA((4,)),
        pltpu.SemaphoreType.DMA((8,)),
    ],
)
def _prop_kernel(g_hbm, src_hbm, dst_hbm, out_hbm,
                 acc, rows, idxb, gsem, isem):
    c = lax.axis_index("c")
    s = lax.axis_index("s")
    wid = c * NS + s
    row_base = pl.multiple_of(s * RPT, 8)

    _zero_fill(rows.at[0], CH, DH)
    _tile_copy(rows.at[0], acc, row_base)
    plsc.subcore_barrier()

    ebase = wid * EPW

    def stage(j, q):
        off = pl.multiple_of(ebase + j * CH, 8)
        pltpu.async_copy(src_hbm.at[pl.ds(off, CH)], idxb.at[q, 0],
                         isem.at[q])
        pltpu.async_copy(dst_hbm.at[pl.ds(off, CH)], idxb.at[q, 1],
                         isem.at[q])

    def wait_stage(q):
        pltpu.make_async_copy(src_hbm.at[pl.ds(0, CH)], idxb.at[q, 0],
                              isem.at[q]).wait()
        pltpu.make_async_copy(src_hbm.at[pl.ds(0, CH)], idxb.at[q, 1],
                              isem.at[q]).wait()

    def gather(b, q):
        pltpu.async_copy(g_hbm.at[idxb.at[q, 0]], rows.at[b], gsem.at[b])

    def wait_gather(b):
        pltpu.make_async_copy(g_hbm.at[idxb.at[0, 0]], rows.at[b],
                              gsem.at[b]).wait()

    def scatter(b, q):
        pltpu.sync_copy(rows.at[b], acc.at[idxb.at[q, 1]], add=True)

    for q in range(8):
        stage(q, q)
    for b in range(4):
        wait_stage(b)
        gather(b, b)

    def body(i, _):
        j8 = i * 8
        for k in range(8):
            j = j8 + k
            b = k % 4
            q = k

            @pl.when(j < NCHUNK)
            def _():
                wait_gather(b)
                scatter(b, q)

            @pl.when(j + 8 < NCHUNK)
            def _():
                stage(j + 8, q)

            @pl.when(j + 4 < NCHUNK)
            def _():
                wait_stage((k + 4) % 8)
                gather(b, (k + 4) % 8)

        return 0

    lax.fori_loop(0, (NCHUNK + 7) // 8, body, 0)
    plsc.subcore_barrier()
    pltpu.sync_copy(acc.at[pl.ds(row_base, RPT)],
                    out_hbm.at[c, pl.ds(row_base, RPT)])


def _pre_body(degp_ref, x_ref, w1_ref, g1_ref, dinv_ref):
    degp = degp_ref[...].astype(jnp.float32)
    deg = 1.0 + degp[0, :NN, 0:1] + degp[1, :NN, 0:1]
    dinv = lax.rsqrt(deg)
    h = jnp.dot(x_ref[...], w1_ref[...], preferred_element_type=jnp.float32)
    g1_ref[...] = h * dinv
    dinv_ref[...] = dinv


def _mid_body(sp_ref, g1_ref, dinv_ref, b1_ref, gamma_ref, beta_ref, g2_ref):
    sp = sp_ref[...]
    dinv = jnp.broadcast_to(dinv_ref[...], (NN, DH))
    p = dinv * (sp[0, :NN] + sp[1, :NN] + g1_ref[...]) + b1_ref[...]
    m = jnp.mean(p, axis=0, keepdims=True)
    v = jnp.mean(p * p, axis=0, keepdims=True) - m * m
    h = (p - m) / jnp.sqrt(v + 1e-5) * gamma_ref[...] + beta_ref[...]
    h = jnp.maximum(h, 0.0)
    g2_ref[...] = h * dinv


def _post_body(sp_ref, g2_ref, dinv_ref, wmu_ref, bmu_ref, wlv_ref, blv_ref,
               zm_ref, zl_ref):
    sp = sp_ref[...]
    q = jnp.broadcast_to(dinv_ref[...], (NN, DH)) * (
        sp[0, :NN] + sp[1, :NN] + g2_ref[...])
    zm_ref[...] = jnp.dot(q, wmu_ref[...],
                          preferred_element_type=jnp.float32) + bmu_ref[...]
    zl_ref[...] = jnp.dot(q, wlv_ref[...],
                          preferred_element_type=jnp.float32) + blv_ref[...]


_pre_call = pl.pallas_call(
    _pre_body,
    out_shape=[jax.ShapeDtypeStruct((NN, DH), jnp.float32),
               jax.ShapeDtypeStruct((NN, 1), jnp.float32)],
)

_mid_call = pl.pallas_call(
    _mid_body,
    out_shape=jax.ShapeDtypeStruct((NN, DH), jnp.float32),
)

_post_call = pl.pallas_call(
    _post_body,
    out_shape=[jax.ShapeDtypeStruct((NN, DL), jnp.float32),
               jax.ShapeDtypeStruct((NN, DL), jnp.float32)],
)


@jax.jit
def kernel(x, edge_index, y, W1, b1, gamma1, beta1, Wmu, bmu, Wlv, blv):
    src_flat = edge_index[:, 0].astype(jnp.int32)
    dst_flat = edge_index[:, 1].astype(jnp.int32)
    dst = dst_flat.reshape(NW, NCHUNK, CH)
    degp = _deg_kernel(dst)
    g1, dinv = _pre_call(degp, x, W1)
    s1 = _prop_kernel(g1, src_flat, dst_flat)
    g2 = _mid_call(s1, g1, dinv, b1.reshape(1, DH), gamma1.reshape(1, DH),
                   beta1.reshape(1, DH))
    s2 = _prop_kernel(g2, src_flat, dst_flat)
    zm, zl = _post_call(s2, g2, dinv, Wmu, bmu.reshape(1, DL), Wlv,
                        blv.reshape(1, DL))
    return (zm, zl)


# speedup vs baseline: 1.0167x; 1.0140x over previous
import functools

import jax
import jax.numpy as jnp
from jax import lax
from jax.experimental import pallas as pl
from jax.experimental.pallas import tpu as pltpu
from jax.experimental.pallas import tpu_sc as plsc

NN = 10000
EE = 320000
DH = 128
DL = 64

NP = 10112
NC, NS = 2, 16
NW = NC * NS
EPW = EE // NW
CH = 80
NCHUNK = EPW // CH
RPT = NP // NS

_mesh = plsc.VectorSubcoreMesh(
    core_axis_name="c", subcore_axis_name="s", num_cores=NC, num_subcores=NS)


def _zero_fill(buf, rows, width):
    zeros16 = jnp.zeros((16,), jnp.float32)

    def zrow(i, _):
        for q in range(width // 16):
            buf[i, pl.ds(q * 16, 16)] = zeros16
        return 0

    lax.fori_loop(0, rows, zrow, 0)


def _tile_copy(src_buf, dst_view, row_base):
    for k in range(RPT // CH):
        pltpu.sync_copy(src_buf, dst_view.at[pl.ds(row_base + k * CH, CH)])
    rem = RPT % CH
    pltpu.sync_copy(src_buf.at[pl.ds(0, rem)],
                    dst_view.at[pl.ds(row_base + (RPT // CH) * CH, rem)])


DEGW = DH


@functools.partial(
    pl.kernel,
    out_type=jax.ShapeDtypeStruct((NC, NP, DEGW), jnp.float32),
    mesh=_mesh,
    scratch_types=[
        pltpu.VMEM_SHARED((NP, DEGW), jnp.float32),
        pltpu.VMEM((NCHUNK, CH), jnp.int32),
        pltpu.VMEM((CH, DEGW), jnp.float32),
        pltpu.SemaphoreType.DMA,
    ],
)
def _deg_kernel(dst_hbm, out_hbm, acc, dst_l, ones_l, ssem):
    c = lax.axis_index("c")
    s = lax.axis_index("s")
    wid = c * NS + s
    row_base = pl.multiple_of(s * RPT, 8)

    pltpu.async_copy(dst_hbm.at[wid], dst_l, ssem)
    _zero_fill(ones_l, CH, DEGW)
    _tile_copy(ones_l, acc, row_base)

    ones16 = jnp.ones((16,), jnp.float32)

    def orow(i, _):
        for q in range(DEGW // 16):
            ones_l[i, pl.ds(q * 16, 16)] = ones16
        return 0

    lax.fori_loop(0, CH, orow, 0)
    pltpu.make_async_copy(dst_hbm.at[0], dst_l, ssem).wait()
    plsc.subcore_barrier()

    def wait_one(_i, _):
        pltpu.make_async_copy(ones_l, acc.at[dst_l.at[0]], ssem).wait()
        return 0

    def body(j, _):
        pltpu.async_copy(ones_l, acc.at[dst_l.at[j]], ssem, add=True)

        @pl.when(j >= 8)
        def _():
            wait_one(0, 0)

        return 0

    lax.fori_loop(0, NCHUNK, body, 0)
    lax.fori_loop(0, 8, wait_one, 0)
    plsc.subcore_barrier()
    pltpu.sync_copy(acc.at[pl.ds(row_base, RPT)],
                    out_hbm.at[c, pl.ds(row_base, RPT)])


@functools.partial(
    pl.kernel,
    out_type=jax.ShapeDtypeStruct((NC, NP, DH), jnp.float32),
    mesh=_mesh,
    scratch_types=[
        pltpu.VMEM_SHARED((NP, DH), jnp.float32),
        pltpu.VMEM((4, CH, DH), jnp.float32),
        pltpu.VMEM((8, 2, CH), jnp.int32),
        pltpu.SemaphoreType.DMA((4,)),
        pltpu.SemaphoreType.DMA((8,)),
    ],
)
def _prop_kernel(g_hbm, src_hbm, dst_hbm, out_hbm,
                 acc, rows, idxb, gsem, isem):
    c = lax.axis_index("c")
    s = lax.axis_index("s")
    wid = c * NS + s
    row_base = pl.multiple_of(s * RPT, 8)


    ebase = wid * EPW

    def stage(j, q):
        off = pl.multiple_of(ebase + j * CH, 8)
        pltpu.async_copy(src_hbm.at[pl.ds(off, CH)], idxb.at[q, 0],
                         isem.at[q])
        pltpu.async_copy(dst_hbm.at[pl.ds(off, CH)], idxb.at[q, 1],
                         isem.at[q])

    def wait_stage(q):
        pltpu.make_async_copy(src_hbm.at[pl.ds(0, CH)], idxb.at[q, 0],
                              isem.at[q]).wait()
        pltpu.make_async_copy(src_hbm.at[pl.ds(0, CH)], idxb.at[q, 1],
                              isem.at[q]).wait()

    def gather(b, q):
        pltpu.async_copy(g_hbm.at[idxb.at[q, 0]], rows.at[b], gsem.at[b])

    def wait_gather(b):
        pltpu.make_async_copy(g_hbm.at[idxb.at[0, 0]], rows.at[b],
                              gsem.at[b]).wait()

    def scatter(b, q):
        pltpu.sync_copy(rows.at[b], acc.at[idxb.at[q, 1]], add=True)

    for q in range(8):
        stage(q, q)
    for b in range(1, 4):
        wait_stage(b)
        gather(b, b)
    _zero_fill(rows.at[0], CH, DH)
    _tile_copy(rows.at[0], acc, row_base)
    wait_stage(0)
    gather(0, 0)
    plsc.subcore_barrier()

    def body(i, _):
        j8 = i * 8
        for k in range(8):
            j = j8 + k
            b = k % 4
            q = k

            @pl.when(j < NCHUNK)
            def _():
                wait_gather(b)
                scatter(b, q)

            @pl.when(j + 8 < NCHUNK)
            def _():
                stage(j + 8, q)

            @pl.when(j + 4 < NCHUNK)
            def _():
                wait_stage((k + 4) % 8)
                gather(b, (k + 4) % 8)

        return 0

    lax.fori_loop(0, (NCHUNK + 7) // 8, body, 0)
    plsc.subcore_barrier()
    pltpu.sync_copy(acc.at[pl.ds(row_base, RPT)],
                    out_hbm.at[c, pl.ds(row_base, RPT)])


def _pre_body(degp_ref, x_ref, w1_ref, g1_ref, dinv_ref):
    degp = degp_ref[...].astype(jnp.float32)
    deg = 1.0 + degp[0, :NN, 0:1] + degp[1, :NN, 0:1]
    dinv = lax.rsqrt(deg)
    h = jnp.dot(x_ref[...], w1_ref[...], preferred_element_type=jnp.float32)
    g1_ref[...] = h * dinv
    dinv_ref[...] = dinv


def _mid_body(sp_ref, g1_ref, dinv_ref, b1_ref, gamma_ref, beta_ref, g2_ref):
    sp = sp_ref[...]
    dinv = jnp.broadcast_to(dinv_ref[...], (NN, DH))
    p = dinv * (sp[0, :NN] + sp[1, :NN] + g1_ref[...]) + b1_ref[...]
    m = jnp.mean(p, axis=0, keepdims=True)
    v = jnp.mean(p * p, axis=0, keepdims=True) - m * m
    h = (p - m) / jnp.sqrt(v + 1e-5) * gamma_ref[...] + beta_ref[...]
    h = jnp.maximum(h, 0.0)
    g2_ref[...] = h * dinv


def _post_body(sp_ref, g2_ref, dinv_ref, wmu_ref, bmu_ref, wlv_ref, blv_ref,
               zm_ref, zl_ref):
    sp = sp_ref[...]
    q = jnp.broadcast_to(dinv_ref[...], (NN, DH)) * (
        sp[0, :NN] + sp[1, :NN] + g2_ref[...])
    zm_ref[...] = jnp.dot(q, wmu_ref[...],
                          preferred_element_type=jnp.float32) + bmu_ref[...]
    zl_ref[...] = jnp.dot(q, wlv_ref[...],
                          preferred_element_type=jnp.float32) + blv_ref[...]


_pre_call = pl.pallas_call(
    _pre_body,
    out_shape=[jax.ShapeDtypeStruct((NN, DH), jnp.float32),
               jax.ShapeDtypeStruct((NN, 1), jnp.float32)],
)

_mid_call = pl.pallas_call(
    _mid_body,
    out_shape=jax.ShapeDtypeStruct((NN, DH), jnp.float32),
)

_post_call = pl.pallas_call(
    _post_body,
    out_shape=[jax.ShapeDtypeStruct((NN, DL), jnp.float32),
               jax.ShapeDtypeStruct((NN, DL), jnp.float32)],
)


@jax.jit
def kernel(x, edge_index, y, W1, b1, gamma1, beta1, Wmu, bmu, Wlv, blv):
    src_flat = edge_index[:, 0].astype(jnp.int32)
    dst_flat = edge_index[:, 1].astype(jnp.int32)
    dst = dst_flat.reshape(NW, NCHUNK, CH)
    degp = _deg_kernel(dst)
    g1, dinv = _pre_call(degp, x, W1)
    s1 = _prop_kernel(g1, src_flat, dst_flat)
    g2 = _mid_call(s1, g1, dinv, b1.reshape(1, DH), gamma1.reshape(1, DH),
                   beta1.reshape(1, DH))
    s2 = _prop_kernel(g2, src_flat, dst_flat)
    zm, zl = _post_call(s2, g2, dinv, Wmu, bmu.reshape(1, DL), Wlv,
                        blv.reshape(1, DL))
    return (zm, zl)
